# Initial kernel scaffold; baseline (speedup 1.0000x reference)
import jax
import jax.numpy as jnp
from jax.experimental import pallas as pl


def kernel(x, pred):
    raise NotImplementedError("write your pallas kernel here")


# speedup vs baseline: 16.8104x; 16.8104x over previous
import jax
import jax.numpy as jnp
from jax.experimental import pallas as pl
from jax.experimental.pallas import tpu as pltpu

NCLS = 80
MAX_DET = 300
CONF = 0.25
IOU_THR = 0.45
NA = 20000
CHUNK = 2048
NCHUNK = (NA + CHUNK - 1) // CHUNK
AP = NCHUNK * CHUNK
NF = 10
NB = 16
NEG = float("-inf")


def _prep_kernel(pred_ref, out_ref):
    j = pl.program_id(1)
    p = pred_ref[0]
    cs = p[4:4 + NCLS]
    score = jnp.max(cs, axis=0, keepdims=True)
    riota = jax.lax.broadcasted_iota(jnp.int32, (NCLS, CHUNK), 0)
    clsi = jnp.min(jnp.where(cs == score, riota, NCLS), axis=0, keepdims=True)
    clsf = clsi.astype(jnp.float32)
    hx = p[2:3] * 0.5
    hy = p[3:4] * 0.5
    bx1 = p[0:1] - hx
    by1 = p[1:2] - hy
    bx2 = p[0:1] + hx
    by2 = p[1:2] + hy
    inv = jnp.float32(1.0 / 640.0)
    nx1 = bx1 * inv + clsf
    ny1 = by1 * inv + clsf
    nx2 = bx2 * inv + clsf
    ny2 = by2 * inv + clsf
    aidx = j * CHUNK + jax.lax.broadcasted_iota(jnp.int32, (1, CHUNK), 1)
    ok = (aidx < NA) & (score > CONF)
    s0 = jnp.where(ok, score, NEG)
    out_ref[0] = jnp.concatenate(
        [s0, clsf, nx1, ny1, nx2, ny2, bx1, by1, bx2, by2], axis=0)


def _nms_kernel(f_ref, out_ref, s_ref):
    s_ref[...] = f_ref[:, 0, :]
    iota_l = jax.lax.broadcasted_iota(jnp.int32, (NB, AP), 1)
    cls = f_ref[:, 1, :]
    nx1 = f_ref[:, 2, :]
    ny1 = f_ref[:, 3, :]
    nx2 = f_ref[:, 4, :]
    ny2 = f_ref[:, 5, :]
    bx1 = f_ref[:, 6, :]
    by1 = f_ref[:, 7, :]
    bx2 = f_ref[:, 8, :]
    by2 = f_ref[:, 9, :]
    area = (jnp.maximum(nx2 - nx1, 0.0) * jnp.maximum(ny2 - ny1, 0.0))

    def body(t, _):
        s = s_ref[...]
        m = jnp.max(s, axis=1, keepdims=True)
        idx = jnp.min(jnp.where(s == m, iota_l, AP), axis=1, keepdims=True)
        oh = iota_l == idx

        def pick(f):
            return jnp.sum(jnp.where(oh, f, 0.0), axis=1, keepdims=True)

        pc = pick(cls)
        px1 = pick(nx1)
        py1 = pick(ny1)
        px2 = pick(nx2)
        py2 = pick(ny2)
        qx1 = pick(bx1)
        qy1 = pick(by1)
        qx2 = pick(bx2)
        qy2 = pick(by2)
        ltx = jnp.maximum(px1, nx1)
        lty = jnp.maximum(py1, ny1)
        rbx = jnp.minimum(px2, nx2)
        rby = jnp.minimum(py2, ny2)
        inter = jnp.maximum(rbx - ltx, 0.0) * jnp.maximum(rby - lty, 0.0)
        parea = jnp.maximum(px2 - px1, 0.0) * jnp.maximum(py2 - py1, 0.0)
        iou = inter / (parea + area - inter + 1e-7)
        supp = (iou > IOU_THR) | oh
        s_ref[...] = jnp.where(supp, NEG, s)

        ok = m > NEG
        km = jnp.where(ok, 1.0, 0.0)
        ps = jnp.where(ok, m, 0.0)
        z = jnp.zeros_like(ps)
        rec = jnp.concatenate(
            [qx1, qy1, qx2, qy2, ps, pc, z, z], axis=1) * km
        out_ref[pl.ds(t, 1)] = rec[None]
        return 0

    jax.lax.fori_loop(0, MAX_DET, body, 0)


def kernel(x, pred):
    del x
    feats = pl.pallas_call(
        _prep_kernel,
        grid=(NB, NCHUNK),
        in_specs=[pl.BlockSpec((1, 4 + NCLS, CHUNK), lambda i, j: (i, 0, j))],
        out_specs=pl.BlockSpec((1, NF, CHUNK), lambda i, j: (i, 0, j)),
        out_shape=jax.ShapeDtypeStruct((NB, NF, AP), jnp.float32),
    )(pred)
    out = pl.pallas_call(
        _nms_kernel,
        out_shape=jax.ShapeDtypeStruct((MAX_DET, NB, 8), jnp.float32),
        scratch_shapes=[pltpu.VMEM((NB, AP), jnp.float32)],
    )(feats)
    return out.transpose(1, 0, 2)[:, :, :6]


# speedup vs baseline: 64.0206x; 3.8084x over previous
import functools

import jax
import jax.numpy as jnp
from jax.experimental import pallas as pl
from jax.experimental.pallas import tpu as pltpu
from jax.experimental.pallas import tpu_sc as plsc

NCLS = 80
MAX_DET = 300
CONF = 0.25
IOU_THR = 0.45
NA = 20000
CHUNK = 2048
NCHUNK = (NA + CHUNK - 1) // CHUNK
AP = NCHUNK * CHUNK
HALF = AP // 2
NF = 5
NB = 16
KCAP = 512
PCAP = 512
PTOT = 2 * PCAP
PMAX = 480
NEG = float("-inf")
INV = 1.0 / 640.0


def _prep_kernel(pred_ref, out_ref):
    j = pl.program_id(1)
    p = pred_ref[0]
    cs = p[4:4 + NCLS]
    score = jnp.max(cs, axis=0, keepdims=True)
    riota = jax.lax.broadcasted_iota(jnp.int32, (NCLS, CHUNK), 0)
    clsi = jnp.min(jnp.where(cs == score, riota, NCLS), axis=0, keepdims=True)
    clsf = clsi.astype(jnp.float32)
    hx = p[2:3] * 0.5
    hy = p[3:4] * 0.5
    inv = jnp.float32(INV)
    nx1 = (p[0:1] - hx) * inv + clsf
    ny1 = (p[1:2] - hy) * inv + clsf
    nx2 = (p[0:1] + hx) * inv + clsf
    ny2 = (p[1:2] + hy) * inv + clsf
    aidx = j * CHUNK + jax.lax.broadcasted_iota(jnp.int32, (1, CHUNK), 1)
    ok = (aidx < NA) & (score > CONF)
    s0 = jnp.where(ok, score, NEG)
    out_ref[0] = jnp.concatenate([s0, nx1, ny1, nx2, ny2], axis=0)


def _tau_kernel(f_ref, tau_ref):
    s = f_ref[:, 0, :]
    iota_l = jax.lax.broadcasted_iota(jnp.int32, (NB, AP), 1)
    in_a = iota_l < HALF

    def body(_, lohi):
        lo, hi = lohi
        mid = (lo + hi) * 0.5
        gt = s > mid
        ca = jnp.sum(jnp.where(gt & in_a, 1.0, 0.0), axis=1, keepdims=True)
        cb = jnp.sum(jnp.where(gt & jnp.logical_not(in_a), 1.0, 0.0),
                     axis=1, keepdims=True)
        fits = jnp.maximum(ca, cb) <= float(PMAX)
        return jnp.where(fits, lo, mid), jnp.where(fits, mid, hi)

    lo0 = jnp.zeros((NB, 1), jnp.float32)
    hi0 = jnp.ones((NB, 1), jnp.float32)
    _, hi = jax.lax.fori_loop(0, 32, body, (lo0, hi0))
    tau_ref[...] = jnp.broadcast_to(hi, (NB, 16))


def _sc_compact(feats_hbm, tau_hbm, pool_hbm, s_v, f1_v, f2_v, f3_v, f4_v,
                tau_v, p0_v, p1_v, p2_v, p3_v, p4_v):
    c = jax.lax.axis_index("c")
    sub = jax.lax.axis_index("s")
    wid = sub * 2 + c
    img = wid // 2
    h = wid % 2
    base = img * NF * AP + h * HALF
    pltpu.sync_copy(feats_hbm.at[pl.ds(base + 0 * AP, HALF)], s_v)
    pltpu.sync_copy(feats_hbm.at[pl.ds(base + 1 * AP, HALF)], f1_v)
    pltpu.sync_copy(feats_hbm.at[pl.ds(base + 2 * AP, HALF)], f2_v)
    pltpu.sync_copy(feats_hbm.at[pl.ds(base + 3 * AP, HALF)], f3_v)
    pltpu.sync_copy(feats_hbm.at[pl.ds(base + 4 * AP, HALF)], f4_v)
    pltpu.sync_copy(tau_hbm.at[pl.ds(img * 16, 16)], tau_v)
    tau = tau_v[...]
    neg = jnp.full((16,), NEG, jnp.float32)
    zero = jnp.zeros((16,), jnp.float32)
    for j in range(PCAP // 16):
        sl0 = pl.ds(j * 16, 16)
        p0_v[sl0] = neg
        p1_v[sl0] = zero
        p2_v[sl0] = zero
        p3_v[sl0] = zero
        p4_v[sl0] = zero

    def body(i, off):
        sl = pl.ds(i * 16, 16)
        sv = s_v[sl]
        msk = sv > tau
        cs = plsc.cumsum(msk.astype(jnp.int32))
        dst = off + cs - 1
        plsc.store_scatter(p0_v, [dst], sv, mask=msk)
        plsc.store_scatter(p1_v, [dst], f1_v[sl], mask=msk)
        plsc.store_scatter(p2_v, [dst], f2_v[sl], mask=msk)
        plsc.store_scatter(p3_v, [dst], f3_v[sl], mask=msk)
        plsc.store_scatter(p4_v, [dst], f4_v[sl], mask=msk)
        return off + jnp.sum(msk.astype(jnp.int32))

    jax.lax.fori_loop(0, HALF // 16, body, jnp.int32(0))
    pbase = (img * 2 + h) * NF * PCAP
    pltpu.sync_copy(p0_v, pool_hbm.at[pl.ds(pbase + 0 * PCAP, PCAP)])
    pltpu.sync_copy(p1_v, pool_hbm.at[pl.ds(pbase + 1 * PCAP, PCAP)])
    pltpu.sync_copy(p2_v, pool_hbm.at[pl.ds(pbase + 2 * PCAP, PCAP)])
    pltpu.sync_copy(p3_v, pool_hbm.at[pl.ds(pbase + 3 * PCAP, PCAP)])
    pltpu.sync_copy(p4_v, pool_hbm.at[pl.ds(pbase + 4 * PCAP, PCAP)])


@functools.cache
def _sc_compact_call():
    return functools.partial(
        pl.kernel,
        mesh=plsc.VectorSubcoreMesh(core_axis_name="c", subcore_axis_name="s"),
        compiler_params=pltpu.CompilerParams(needs_layout_passes=False),
        out_type=jax.ShapeDtypeStruct((NB * 2 * NF * PCAP,), jnp.float32),
        scratch_types=[
            pltpu.VMEM((HALF,), jnp.float32),
            pltpu.VMEM((HALF,), jnp.float32),
            pltpu.VMEM((HALF,), jnp.float32),
            pltpu.VMEM((HALF,), jnp.float32),
            pltpu.VMEM((HALF,), jnp.float32),
            pltpu.VMEM((16,), jnp.float32),
            pltpu.VMEM((PCAP,), jnp.float32),
            pltpu.VMEM((PCAP,), jnp.float32),
            pltpu.VMEM((PCAP,), jnp.float32),
            pltpu.VMEM((PCAP,), jnp.float32),
            pltpu.VMEM((PCAP,), jnp.float32),
        ],
    )(_sc_compact)


def _nms_kernel(f_ref, tau_ref, pool_ref, out_ref, s_ref, sp_ref, pf_ref,
                kb_ref):
    out_ref[...] = jnp.zeros((6, NB, KCAP), jnp.float32)
    kb_ref[0:4] = jnp.full((4, NB, KCAP), 4096.0, jnp.float32)
    kb_ref[4:5] = jnp.zeros((1, NB, KCAP), jnp.float32)
    sp_ref[:, 0:PCAP] = pool_ref[:, 0, 0, :]
    sp_ref[:, PCAP:PTOT] = pool_ref[:, 1, 0, :]
    for k in range(4):
        pf_ref[k, :, 0:PCAP] = pool_ref[:, 0, k + 1, :]
        pf_ref[k, :, PCAP:PTOT] = pool_ref[:, 1, k + 1, :]

    iota_p = jax.lax.broadcasted_iota(jnp.int32, (NB, PTOT), 1)
    iota_l = jax.lax.broadcasted_iota(jnp.int32, (NB, AP), 1)
    iota_k = jax.lax.broadcasted_iota(jnp.int32, (NB, KCAP), 1)

    def check_and_append(cnt, m, px1, py1, px2, py2):
        pcls = jnp.floor(px1 + jnp.float32(INV))
        pa = jnp.maximum(px2 - px1, 0.0) * jnp.maximum(py2 - py1, 0.0)
        kx1 = kb_ref[0]
        ky1 = kb_ref[1]
        kx2 = kb_ref[2]
        ky2 = kb_ref[3]
        ka = kb_ref[4]
        inter = (jnp.maximum(jnp.minimum(px2, kx2) - jnp.maximum(px1, kx1),
                             0.0)
                 * jnp.maximum(jnp.minimum(py2, ky2) - jnp.maximum(py1, ky1),
                               0.0))
        iou = inter / (ka + pa - inter + 1e-7)
        sup = jnp.max(jnp.where(iou > IOU_THR, 1.0, 0.0), axis=1,
                      keepdims=True) > 0.0
        accept = (m > NEG) & jnp.logical_not(sup) & (cnt < MAX_DET)
        ohk = (iota_k == cnt) & accept
        kb_ref[0] = jnp.where(ohk, px1, kx1)
        kb_ref[1] = jnp.where(ohk, py1, ky1)
        kb_ref[2] = jnp.where(ohk, px2, kx2)
        kb_ref[3] = jnp.where(ohk, py2, ky2)
        kb_ref[4] = jnp.where(ohk, pa, ka)
        out_ref[0] = jnp.where(ohk, (px1 - pcls) * 640.0, out_ref[0])
        out_ref[1] = jnp.where(ohk, (py1 - pcls) * 640.0, out_ref[1])
        out_ref[2] = jnp.where(ohk, (px2 - pcls) * 640.0, out_ref[2])
        out_ref[3] = jnp.where(ohk, (py2 - pcls) * 640.0, out_ref[3])
        out_ref[4] = jnp.where(ohk, m, out_ref[4])
        out_ref[5] = jnp.where(ohk, pcls, out_ref[5])
        return cnt + accept.astype(jnp.int32)

    def cond1(carry):
        _, alive = carry
        return alive

    def body1(carry):
        cnt, _ = carry
        s = sp_ref[...]
        m = jnp.max(s, axis=1, keepdims=True)
        idx = jnp.min(jnp.where(s == m, iota_p, PTOT), axis=1, keepdims=True)
        oh = iota_p == idx
        sp_ref[...] = jnp.where(oh, NEG, s)

        def pick(r):
            return jnp.sum(jnp.where(oh, r, 0.0), axis=1, keepdims=True)

        cnt = check_and_append(cnt, m, pick(pf_ref[0]), pick(pf_ref[1]),
                               pick(pf_ref[2]), pick(pf_ref[3]))
        alive = jnp.any((cnt < MAX_DET) & (m > NEG))
        return cnt, alive

    cnt, _ = jax.lax.while_loop(
        cond1, body1, (jnp.zeros((NB, 1), jnp.int32), jnp.bool_(True)))

    tau = tau_ref[:, 0:1]
    s_full = f_ref[:, 0, :]
    s_ref[...] = jnp.where(s_full > tau, NEG, s_full)

    def cond2(carry):
        _, alive = carry
        return alive

    def body2(carry):
        cnt, _ = carry
        s = s_ref[...]
        m = jnp.max(s, axis=1, keepdims=True)
        idx = jnp.min(jnp.where(s == m, iota_l, AP), axis=1, keepdims=True)
        oh = iota_l == idx
        s_ref[...] = jnp.where(oh, NEG, s)

        def pick(r):
            return jnp.sum(jnp.where(oh, r, 0.0), axis=1, keepdims=True)

        cnt = check_and_append(cnt, m, pick(f_ref[:, 1, :]),
                               pick(f_ref[:, 2, :]), pick(f_ref[:, 3, :]),
                               pick(f_ref[:, 4, :]))
        alive = jnp.any((cnt < MAX_DET) & (m > NEG))
        return cnt, alive

    alive0 = jnp.any(cnt < MAX_DET)
    jax.lax.while_loop(cond2, body2, (cnt, alive0))


def kernel(x, pred):
    del x
    feats = pl.pallas_call(
        _prep_kernel,
        grid=(NB, NCHUNK),
        in_specs=[pl.BlockSpec((1, 4 + NCLS, CHUNK), lambda i, j: (i, 0, j))],
        out_specs=pl.BlockSpec((1, NF, CHUNK), lambda i, j: (i, 0, j)),
        out_shape=jax.ShapeDtypeStruct((NB, NF, AP), jnp.float32),
    )(pred)
    tau = pl.pallas_call(
        _tau_kernel,
        out_shape=jax.ShapeDtypeStruct((NB, 16), jnp.float32),
    )(feats)
    pool = _sc_compact_call()(feats.reshape(-1), tau.reshape(-1))
    pool = pool.reshape(NB, 2, NF, PCAP)
    res = pl.pallas_call(
        _nms_kernel,
        out_shape=jax.ShapeDtypeStruct((6, NB, KCAP), jnp.float32),
        scratch_shapes=[pltpu.VMEM((NB, AP), jnp.float32),
                        pltpu.VMEM((NB, PTOT), jnp.float32),
                        pltpu.VMEM((4, NB, PTOT), jnp.float32),
                        pltpu.VMEM((NF, NB, KCAP), jnp.float32)],
    )(feats, tau, pool)
    return res.transpose(1, 2, 0)[:, :MAX_DET, :]


# speedup vs baseline: 70.0055x; 1.0935x over previous
import functools

import jax
import jax.numpy as jnp
from jax.experimental import pallas as pl
from jax.experimental.pallas import tpu as pltpu
from jax.experimental.pallas import tpu_sc as plsc

NCLS = 80
MAX_DET = 300
CONF = 0.25
IOU_THR = 0.45
NA = 20000
CHUNK = 2048
NCHUNK = (NA + CHUNK - 1) // CHUNK
AP = NCHUNK * CHUNK
HALF = AP // 2
NF = 5
NB = 16
KCAP = 512
PCAP = 512
PTOT = 2 * PCAP
PMAX = 480
NEG = float("-inf")
INV = 1.0 / 640.0


def _prep_kernel(pred_ref, out_ref):
    j = pl.program_id(1)
    p = pred_ref[0]
    cs = p[4:4 + NCLS]
    score = jnp.max(cs, axis=0, keepdims=True)
    riota = jax.lax.broadcasted_iota(jnp.int32, (NCLS, CHUNK), 0)
    clsi = jnp.min(jnp.where(cs == score, riota, NCLS), axis=0, keepdims=True)
    clsf = clsi.astype(jnp.float32)
    hx = p[2:3] * 0.5
    hy = p[3:4] * 0.5
    inv = jnp.float32(INV)
    nx1 = (p[0:1] - hx) * inv + clsf
    ny1 = (p[1:2] - hy) * inv + clsf
    nx2 = (p[0:1] + hx) * inv + clsf
    ny2 = (p[1:2] + hy) * inv + clsf
    aidx = j * CHUNK + jax.lax.broadcasted_iota(jnp.int32, (1, CHUNK), 1)
    ok = (aidx < NA) & (score > CONF)
    s0 = jnp.where(ok, score, NEG)
    out_ref[0] = jnp.concatenate([s0, nx1, ny1, nx2, ny2], axis=0)


def _tau_kernel(f_ref, tau_ref):
    s = f_ref[:, 0, :]
    iota_l = jax.lax.broadcasted_iota(jnp.int32, (NB, AP), 1)
    in_a = iota_l < HALF

    def body(_, lohi):
        lo, hi = lohi
        mid = (lo + hi) * 0.5
        gt = s > mid
        ca = jnp.sum(jnp.where(gt & in_a, 1.0, 0.0), axis=1, keepdims=True)
        cb = jnp.sum(jnp.where(gt & jnp.logical_not(in_a), 1.0, 0.0),
                     axis=1, keepdims=True)
        fits = jnp.maximum(ca, cb) <= float(PMAX)
        return jnp.where(fits, lo, mid), jnp.where(fits, mid, hi)

    lo0 = jnp.zeros((NB, 1), jnp.float32)
    hi0 = jnp.ones((NB, 1), jnp.float32)
    _, hi = jax.lax.fori_loop(0, 32, body, (lo0, hi0))
    tau_ref[...] = jnp.broadcast_to(hi, (NB, 16))


def _sc_compact(feats_hbm, tau_hbm, pool_hbm, s_v, f1_v, f2_v, f3_v, f4_v,
                tau_v, p0_v, p1_v, p2_v, p3_v, p4_v):
    c = jax.lax.axis_index("c")
    sub = jax.lax.axis_index("s")
    wid = sub * 2 + c
    img = wid // 2
    h = wid % 2
    base = img * NF * AP + h * HALF
    pltpu.sync_copy(feats_hbm.at[pl.ds(base + 0 * AP, HALF)], s_v)
    pltpu.sync_copy(feats_hbm.at[pl.ds(base + 1 * AP, HALF)], f1_v)
    pltpu.sync_copy(feats_hbm.at[pl.ds(base + 2 * AP, HALF)], f2_v)
    pltpu.sync_copy(feats_hbm.at[pl.ds(base + 3 * AP, HALF)], f3_v)
    pltpu.sync_copy(feats_hbm.at[pl.ds(base + 4 * AP, HALF)], f4_v)
    pltpu.sync_copy(tau_hbm.at[pl.ds(img * 16, 16)], tau_v)
    tau = tau_v[...]
    neg = jnp.full((16,), NEG, jnp.float32)
    zero = jnp.zeros((16,), jnp.float32)
    for j in range(PCAP // 16):
        sl0 = pl.ds(j * 16, 16)
        p0_v[sl0] = neg
        p1_v[sl0] = zero
        p2_v[sl0] = zero
        p3_v[sl0] = zero
        p4_v[sl0] = zero

    def body(i, off):
        sl = pl.ds(i * 16, 16)
        sv = s_v[sl]
        msk = sv > tau
        cs = plsc.cumsum(msk.astype(jnp.int32))
        dst = off + cs - 1
        plsc.store_scatter(p0_v, [dst], sv, mask=msk)
        plsc.store_scatter(p1_v, [dst], f1_v[sl], mask=msk)
        plsc.store_scatter(p2_v, [dst], f2_v[sl], mask=msk)
        plsc.store_scatter(p3_v, [dst], f3_v[sl], mask=msk)
        plsc.store_scatter(p4_v, [dst], f4_v[sl], mask=msk)
        return off + jnp.sum(msk.astype(jnp.int32))

    jax.lax.fori_loop(0, HALF // 16, body, jnp.int32(0))
    pbase = (img * 2 + h) * NF * PCAP
    pltpu.sync_copy(p0_v, pool_hbm.at[pl.ds(pbase + 0 * PCAP, PCAP)])
    pltpu.sync_copy(p1_v, pool_hbm.at[pl.ds(pbase + 1 * PCAP, PCAP)])
    pltpu.sync_copy(p2_v, pool_hbm.at[pl.ds(pbase + 2 * PCAP, PCAP)])
    pltpu.sync_copy(p3_v, pool_hbm.at[pl.ds(pbase + 3 * PCAP, PCAP)])
    pltpu.sync_copy(p4_v, pool_hbm.at[pl.ds(pbase + 4 * PCAP, PCAP)])


@functools.cache
def _sc_compact_call():
    return functools.partial(
        pl.kernel,
        mesh=plsc.VectorSubcoreMesh(core_axis_name="c", subcore_axis_name="s"),
        compiler_params=pltpu.CompilerParams(needs_layout_passes=False),
        out_type=jax.ShapeDtypeStruct((NB * 2 * NF * PCAP,), jnp.float32),
        scratch_types=[
            pltpu.VMEM((HALF,), jnp.float32),
            pltpu.VMEM((HALF,), jnp.float32),
            pltpu.VMEM((HALF,), jnp.float32),
            pltpu.VMEM((HALF,), jnp.float32),
            pltpu.VMEM((HALF,), jnp.float32),
            pltpu.VMEM((16,), jnp.float32),
            pltpu.VMEM((PCAP,), jnp.float32),
            pltpu.VMEM((PCAP,), jnp.float32),
            pltpu.VMEM((PCAP,), jnp.float32),
            pltpu.VMEM((PCAP,), jnp.float32),
            pltpu.VMEM((PCAP,), jnp.float32),
        ],
    )(_sc_compact)


def _nms_kernel(f_ref, tau_ref, pool_ref, out_ref, s_ref, sp_ref, pf_ref,
                kb_ref):
    out_ref[...] = jnp.zeros((6, NB, KCAP), jnp.float32)
    kb_ref[0:4] = jnp.full((4, NB, KCAP), 4096.0, jnp.float32)
    kb_ref[4:5] = jnp.zeros((1, NB, KCAP), jnp.float32)
    sp_ref[:, 0:PCAP] = pool_ref[:, 0, 0, :]
    sp_ref[:, PCAP:PTOT] = pool_ref[:, 1, 0, :]
    for k in range(4):
        pf_ref[k, :, 0:PCAP] = pool_ref[:, 0, k + 1, :]
        pf_ref[k, :, PCAP:PTOT] = pool_ref[:, 1, k + 1, :]

    iota_p = jax.lax.broadcasted_iota(jnp.int32, (NB, PTOT), 1)
    iota_l = jax.lax.broadcasted_iota(jnp.int32, (NB, AP), 1)
    iota_k = jax.lax.broadcasted_iota(jnp.int32, (NB, KCAP), 1)

    def check_and_append(cnt, m, px1, py1, px2, py2):
        pcls = jnp.floor(px1 + jnp.float32(INV))
        pa = jnp.maximum(px2 - px1, 0.0) * jnp.maximum(py2 - py1, 0.0)
        kx1 = kb_ref[0]
        ky1 = kb_ref[1]
        kx2 = kb_ref[2]
        ky2 = kb_ref[3]
        ka = kb_ref[4]
        inter = (jnp.maximum(jnp.minimum(px2, kx2) - jnp.maximum(px1, kx1),
                             0.0)
                 * jnp.maximum(jnp.minimum(py2, ky2) - jnp.maximum(py1, ky1),
                               0.0))
        iou = inter / (ka + pa - inter + 1e-7)
        sup = jnp.max(jnp.where(iou > IOU_THR, 1.0, 0.0), axis=1,
                      keepdims=True) > 0.0
        accept = (m > NEG) & jnp.logical_not(sup) & (cnt < MAX_DET)
        ohk = (iota_k == cnt) & accept
        kb_ref[0] = jnp.where(ohk, px1, kx1)
        kb_ref[1] = jnp.where(ohk, py1, ky1)
        kb_ref[2] = jnp.where(ohk, px2, kx2)
        kb_ref[3] = jnp.where(ohk, py2, ky2)
        kb_ref[4] = jnp.where(ohk, pa, ka)
        out_ref[0] = jnp.where(ohk, (px1 - pcls) * 640.0, out_ref[0])
        out_ref[1] = jnp.where(ohk, (py1 - pcls) * 640.0, out_ref[1])
        out_ref[2] = jnp.where(ohk, (px2 - pcls) * 640.0, out_ref[2])
        out_ref[3] = jnp.where(ohk, (py2 - pcls) * 640.0, out_ref[3])
        out_ref[4] = jnp.where(ohk, m, out_ref[4])
        out_ref[5] = jnp.where(ohk, pcls, out_ref[5])
        return cnt + accept.astype(jnp.int32)

    def cond1(carry):
        _, alive = carry
        return alive

    def kept_sup(px1, py1, px2, py2, pa):
        inter = (jnp.maximum(jnp.minimum(px2, kb_ref[2]) -
                             jnp.maximum(px1, kb_ref[0]), 0.0)
                 * jnp.maximum(jnp.minimum(py2, kb_ref[3]) -
                               jnp.maximum(py1, kb_ref[1]), 0.0))
        iou = inter / (kb_ref[4] + pa - inter + 1e-7)
        return jnp.max(jnp.where(iou > IOU_THR, 1.0, 0.0), axis=1,
                       keepdims=True) > 0.0

    def append(cnt, accept, m, px1, py1, px2, py2, pa):
        pcls = jnp.floor(px1 + jnp.float32(INV))
        ohk = (iota_k == cnt) & accept
        kb_ref[0] = jnp.where(ohk, px1, kb_ref[0])
        kb_ref[1] = jnp.where(ohk, py1, kb_ref[1])
        kb_ref[2] = jnp.where(ohk, px2, kb_ref[2])
        kb_ref[3] = jnp.where(ohk, py2, kb_ref[3])
        kb_ref[4] = jnp.where(ohk, pa, kb_ref[4])
        out_ref[0] = jnp.where(ohk, (px1 - pcls) * 640.0, out_ref[0])
        out_ref[1] = jnp.where(ohk, (py1 - pcls) * 640.0, out_ref[1])
        out_ref[2] = jnp.where(ohk, (px2 - pcls) * 640.0, out_ref[2])
        out_ref[3] = jnp.where(ohk, (py2 - pcls) * 640.0, out_ref[3])
        out_ref[4] = jnp.where(ohk, m, out_ref[4])
        out_ref[5] = jnp.where(ohk, pcls, out_ref[5])
        return cnt + accept.astype(jnp.int32)

    def body1(carry):
        cnt, _ = carry
        s = sp_ref[...]
        m1 = jnp.max(s, axis=1, keepdims=True)
        i1 = jnp.min(jnp.where(s == m1, iota_p, PTOT), axis=1, keepdims=True)
        oh1 = iota_p == i1
        s2 = jnp.where(oh1, NEG, s)
        m2 = jnp.max(s2, axis=1, keepdims=True)
        i2 = jnp.min(jnp.where(s2 == m2, iota_p, PTOT), axis=1, keepdims=True)
        oh2 = iota_p == i2
        sp_ref[...] = jnp.where(oh1 | oh2, NEG, s)

        def pick(r, oh):
            return jnp.sum(jnp.where(oh, r, 0.0), axis=1, keepdims=True)

        ax1 = pick(pf_ref[0], oh1)
        ay1 = pick(pf_ref[1], oh1)
        ax2 = pick(pf_ref[2], oh1)
        ay2 = pick(pf_ref[3], oh1)
        bx1 = pick(pf_ref[0], oh2)
        by1 = pick(pf_ref[1], oh2)
        bx2 = pick(pf_ref[2], oh2)
        by2 = pick(pf_ref[3], oh2)
        aa = jnp.maximum(ax2 - ax1, 0.0) * jnp.maximum(ay2 - ay1, 0.0)
        ba = jnp.maximum(bx2 - bx1, 0.0) * jnp.maximum(by2 - by1, 0.0)
        supa = kept_sup(ax1, ay1, ax2, ay2, aa)
        supb = kept_sup(bx1, by1, bx2, by2, ba)
        acc1 = (m1 > NEG) & jnp.logical_not(supa) & (cnt < MAX_DET)
        iab = (jnp.maximum(jnp.minimum(bx2, ax2) - jnp.maximum(bx1, ax1), 0.0)
               * jnp.maximum(jnp.minimum(by2, ay2) - jnp.maximum(by1, ay1),
                             0.0))
        iou_ab = iab / (aa + ba - iab + 1e-7)
        supb = supb | (acc1 & (iou_ab > IOU_THR))
        cnt = append(cnt, acc1, m1, ax1, ay1, ax2, ay2, aa)
        acc2 = (m2 > NEG) & jnp.logical_not(supb) & (cnt < MAX_DET)
        cnt = append(cnt, acc2, m2, bx1, by1, bx2, by2, ba)
        alive = jnp.any((cnt < MAX_DET) & (m2 > NEG))
        return cnt, alive

    cnt, _ = jax.lax.while_loop(
        cond1, body1, (jnp.zeros((NB, 1), jnp.int32), jnp.bool_(True)))

    tau = tau_ref[:, 0:1]
    s_full = f_ref[:, 0, :]
    s_ref[...] = jnp.where(s_full > tau, NEG, s_full)

    def cond2(carry):
        _, alive = carry
        return alive

    def body2(carry):
        cnt, _ = carry
        s = s_ref[...]
        m = jnp.max(s, axis=1, keepdims=True)
        idx = jnp.min(jnp.where(s == m, iota_l, AP), axis=1, keepdims=True)
        oh = iota_l == idx
        s_ref[...] = jnp.where(oh, NEG, s)

        def pick(r):
            return jnp.sum(jnp.where(oh, r, 0.0), axis=1, keepdims=True)

        cnt = check_and_append(cnt, m, pick(f_ref[:, 1, :]),
                               pick(f_ref[:, 2, :]), pick(f_ref[:, 3, :]),
                               pick(f_ref[:, 4, :]))
        alive = jnp.any((cnt < MAX_DET) & (m > NEG))
        return cnt, alive

    alive0 = jnp.any(cnt < MAX_DET)
    jax.lax.while_loop(cond2, body2, (cnt, alive0))


def kernel(x, pred):
    del x
    feats = pl.pallas_call(
        _prep_kernel,
        grid=(NB, NCHUNK),
        in_specs=[pl.BlockSpec((1, 4 + NCLS, CHUNK), lambda i, j: (i, 0, j))],
        out_specs=pl.BlockSpec((1, NF, CHUNK), lambda i, j: (i, 0, j)),
        out_shape=jax.ShapeDtypeStruct((NB, NF, AP), jnp.float32),
    )(pred)
    tau = pl.pallas_call(
        _tau_kernel,
        out_shape=jax.ShapeDtypeStruct((NB, 16), jnp.float32),
    )(feats)
    pool = _sc_compact_call()(feats.reshape(-1), tau.reshape(-1))
    pool = pool.reshape(NB, 2, NF, PCAP)
    res = pl.pallas_call(
        _nms_kernel,
        out_shape=jax.ShapeDtypeStruct((6, NB, KCAP), jnp.float32),
        scratch_shapes=[pltpu.VMEM((NB, AP), jnp.float32),
                        pltpu.VMEM((NB, PTOT), jnp.float32),
                        pltpu.VMEM((4, NB, PTOT), jnp.float32),
                        pltpu.VMEM((NF, NB, KCAP), jnp.float32)],
    )(feats, tau, pool)
    return res.transpose(1, 2, 0)[:, :MAX_DET, :]
